# Initial kernel scaffold; baseline (speedup 1.0000x reference)
import jax
import jax.numpy as jnp
from jax.experimental import pallas as pl


def kernel(inputs, W):
    raise NotImplementedError("write your pallas kernel here")


# speedup vs baseline: 8.4909x; 8.4909x over previous
import functools

import jax
import jax.numpy as jnp
from jax import lax
from jax.experimental import pallas as pl
from jax.experimental.pallas import tpu as pltpu
from jax.experimental.pallas import tpu_sc as plsc


def _scores_argmin_body(x_ref, w_ref, a_ref, b_ref, idx_ref,
                        best_val, best_idx, *, kk_size, n_k):
    kk = pl.program_id(0)
    tt = pl.program_id(1)
    t_size = x_ref.shape[0]

    dot = lax.dot_general(x_ref[...], w_ref[...],
                          (((1,), (1,)), ((), ())),
                          preferred_element_type=jnp.float32)
    d = (a_ref[...] + b_ref[...]) - 2.0 * dot

    local_min = jnp.min(d, axis=1, keepdims=True)
    lanes = lax.broadcasted_iota(jnp.int32, d.shape, 1)
    big = jnp.int32(2 ** 30)
    local_arg = jnp.min(jnp.where(d == local_min, lanes, big),
                        axis=1, keepdims=True) + kk * kk_size

    row = tt * t_size

    @pl.when(kk == 0)
    def _():
        best_val[pl.ds(row, t_size), :] = local_min
        best_idx[pl.ds(row, t_size), :] = local_arg

    @pl.when(kk > 0)
    def _():
        prev_v = best_val[pl.ds(row, t_size), :]
        prev_i = best_idx[pl.ds(row, t_size), :]
        better = local_min < prev_v
        best_val[pl.ds(row, t_size), :] = jnp.where(better, local_min, prev_v)
        best_idx[pl.ds(row, t_size), :] = jnp.where(better, local_arg, prev_i)

    idx_ref[...] = best_idx[pl.ds(row, t_size), :]


def _scores_argmin(x, W, a, b, *, t_size=512, kk_size=2048):
    BT, D = x.shape
    K = W.shape[0]
    n_t = BT // t_size
    n_k = K // kk_size
    body = functools.partial(_scores_argmin_body, kk_size=kk_size, n_k=n_k)
    return pl.pallas_call(
        body,
        grid=(n_k, n_t),
        in_specs=[
            pl.BlockSpec((t_size, D), lambda k, t: (t, 0)),
            pl.BlockSpec((kk_size, D), lambda k, t: (k, 0)),
            pl.BlockSpec((t_size, 1), lambda k, t: (t, 0)),
            pl.BlockSpec((1, kk_size), lambda k, t: (0, k)),
        ],
        out_specs=pl.BlockSpec((t_size, 1), lambda k, t: (t, 0)),
        out_shape=jax.ShapeDtypeStruct((BT, 1), jnp.int32),
        scratch_shapes=[
            pltpu.VMEM((BT, 1), jnp.float32),
            pltpu.VMEM((BT, 1), jnp.int32),
        ],
    )(x, W, a, b)


def _sc_gather(W, idx):
    BT = idx.shape[0]
    D = W.shape[1]
    info = plsc.get_sparse_core_info()
    NW = info.num_cores * info.num_subcores
    b_per_w = BT // NW
    chunk = 128
    n_chunks = b_per_w // chunk
    mesh = plsc.VectorSubcoreMesh(core_axis_name="c", subcore_axis_name="s")

    @functools.partial(
        pl.kernel,
        mesh=mesh,
        out_type=jax.ShapeDtypeStruct((BT, D), jnp.float32),
        scratch_types=[
            pltpu.VMEM((chunk,), jnp.int32),
            pltpu.VMEM((chunk, D), jnp.float32),
            pltpu.SemaphoreType.DMA,
        ],
    )
    def gather_kernel(w_hbm, idx_hbm, out_hbm, idx_v, rows_v, sem):
        wid = lax.axis_index("s") * info.num_cores + lax.axis_index("c")
        base = wid * b_per_w
        for c in range(n_chunks):
            off = base + c * chunk
            pltpu.sync_copy(idx_hbm.at[pl.ds(off, chunk)], idx_v)
            pltpu.async_copy(w_hbm.at[idx_v], rows_v, sem).wait()
            pltpu.sync_copy(rows_v, out_hbm.at[pl.ds(off, chunk)])

    return gather_kernel(W, idx)


def _st_loss_body(x_ref, q_ref, st_ref, loss_ref):
    i = pl.program_id(0)
    t = q_ref[...] - x_ref[...]
    st_ref[...] = x_ref[...] + t

    @pl.when(i == 0)
    def _():
        loss_ref[...] = jnp.zeros_like(loss_ref)

    loss_ref[...] += jnp.sum(t * t, axis=(0, 1), keepdims=True)


def _st_loss(x, q, *, t_size=1024):
    BT, D = x.shape
    n_t = BT // t_size
    return pl.pallas_call(
        _st_loss_body,
        grid=(n_t,),
        in_specs=[
            pl.BlockSpec((t_size, D), lambda t: (t, 0)),
            pl.BlockSpec((t_size, D), lambda t: (t, 0)),
        ],
        out_specs=[
            pl.BlockSpec((t_size, D), lambda t: (t, 0)),
            pl.BlockSpec((1, 1), lambda t: (0, 0)),
        ],
        out_shape=[
            jax.ShapeDtypeStruct((BT, D), jnp.float32),
            jax.ShapeDtypeStruct((1, 1), jnp.float32),
        ],
    )(x, q)


def kernel(inputs, W):
    B, T, D = inputs.shape
    K = W.shape[0]
    BT = B * T

    x = inputs.reshape(BT, D)
    a = jnp.sum(inputs ** 2, axis=2, keepdims=True).reshape(BT, 1)
    b = jnp.sum(W ** 2, axis=1).reshape(1, K)

    idx = _scores_argmin(x, W, a, b).reshape(BT)
    q = _sc_gather(W, idx)
    st, loss_sum = _st_loss(x, q)

    m = loss_sum[0, 0] / (B * T * D)
    loss = m + 0.25 * m
    return loss, st.reshape(B, T, D)
